# baseline (speedup 1.0000x reference)
import functools

import jax
import jax.numpy as jnp
from jax import lax
from jax.experimental import pallas as pl
from jax.experimental.pallas import tpu as pltpu
from jax.experimental.pallas import tpu_sc as plsc

_NC = 2
_NS = 16
_NW = _NC * _NS
_LANES = 128
_NBUF = 4


@functools.lru_cache(maxsize=None)
def _build(vocab, dim, nrows, hist):
    n_blocks = nrows // _LANES
    blocks_per_w = n_blocks // _NW
    b_per_w = nrows // _NW
    n_chunks = hist * blocks_per_w
    fgroups = dim // 8

    mesh = plsc.VectorSubcoreMesh(
        core_axis_name="c", subcore_axis_name="s",
        num_cores=_NC, num_subcores=_NS)

    @functools.partial(
        pl.kernel,
        out_type=jax.ShapeDtypeStruct(
            (hist, fgroups, n_blocks, 8, _LANES, 1), jnp.float32),
        mesh=mesh,
        compiler_params=pltpu.CompilerParams(
            use_tc_tiling_on_sc=False, needs_layout_passes=False),
        scratch_types=[
            pltpu.VMEM((hist, b_per_w), jnp.int32),
            [pltpu.VMEM((_LANES, dim), jnp.float32)] * _NBUF,
            [pltpu.SemaphoreType.DMA] * _NBUF,
            [pltpu.SemaphoreType.DMA] * _NBUF,
        ],
    )
    def gather_kernel(table_hbm, idx_hbm, out_hbm, idx_blk, row_bufs,
                      gsems, ssems):
        wid = lax.axis_index("s") * _NC + lax.axis_index("c")
        b0 = wid * b_per_w
        bb0 = wid * blocks_per_w

        pltpu.sync_copy(idx_hbm.at[:, pl.ds(b0, b_per_w)], idx_blk)

        def drain_stores(b):
            for f in range(dim):
                pltpu.make_async_copy(
                    row_bufs[b].at[:, pl.ds(f, 1)],
                    out_hbm.at[0, f // 8, 0, f % 8], ssems[b]).wait()

        def start(t, b, drain):
            if drain:
                drain_stores(b)
            h = t // blocks_per_w
            bbl = t % blocks_per_w
            pltpu.async_copy(
                table_hbm.at[idx_blk.at[h, pl.ds(bbl * _LANES, _LANES)]],
                row_bufs[b], gsems[b])

        def finish(t, b):
            h = t // blocks_per_w
            bb = bb0 + t % blocks_per_w
            pltpu.make_async_copy(
                table_hbm.at[idx_blk.at[h, pl.ds(0, _LANES)]],
                row_bufs[b], gsems[b]).wait()
            for f in range(dim):
                pltpu.async_copy(
                    row_bufs[b].at[:, pl.ds(f, 1)],
                    out_hbm.at[h, f // 8, bb, f % 8], ssems[b])

        start(0, 0, False)
        start(1, 1, False)
        start(2, 2, False)
        finish(0, 0)
        start(3, 3, False)
        finish(1, 1)
        start(4, 0, True)
        finish(2, 2)
        start(5, 1, True)
        finish(3, 3)
        start(6, 2, True)

        def body(i, carry):
            for s in range(4):
                t = 4 + i * 4 + s
                finish(t, s)
                start(t + 3, (s + 3) % 4, True)
            return carry

        lax.fori_loop(0, (n_chunks - 8) // 4, body, 0)

        finish(n_chunks - 4, (n_chunks - 4) % 4)
        start(n_chunks - 1, (n_chunks - 1) % 4, True)
        finish(n_chunks - 3, (n_chunks - 3) % 4)
        finish(n_chunks - 2, (n_chunks - 2) % 4)
        finish(n_chunks - 1, (n_chunks - 1) % 4)
        for b in range(_NBUF):
            drain_stores(b)

    return gather_kernel


def kernel(indices, table):
    nrows, hist = indices.shape
    vocab, dim = table.shape
    out6 = _build(vocab, dim, nrows, hist)(table, indices.T)
    return out6.transpose(2, 4, 0, 1, 3, 5).reshape(nrows, hist, dim)


# speedup vs baseline: 68.3914x; 68.3914x over previous
import functools

import jax
import jax.numpy as jnp
from jax import lax
from jax.experimental import pallas as pl
from jax.experimental.pallas import tpu as pltpu
from jax.experimental.pallas import tpu_sc as plsc

_NC = 2
_NS = 16
_NW = _NC * _NS
_LANES = 128
_NBUF = 2


@functools.lru_cache(maxsize=None)
def _build(vocab, dim, nrows, hist):
    n_blocks = nrows // _LANES
    blocks_per_w = n_blocks // _NW
    b_per_w = nrows // _NW
    n_chunks = hist * blocks_per_w
    fgroups = dim // 8

    mesh = plsc.VectorSubcoreMesh(
        core_axis_name="c", subcore_axis_name="s",
        num_cores=_NC, num_subcores=_NS)

    @functools.partial(
        pl.kernel,
        out_type=jax.ShapeDtypeStruct(
            (hist, fgroups, n_blocks, 8, _LANES), jnp.float32),
        mesh=mesh,
        compiler_params=pltpu.CompilerParams(
            use_tc_tiling_on_sc=False, needs_layout_passes=False),
        scratch_types=[
            pltpu.VMEM((hist, b_per_w), jnp.int32),
            [pltpu.VMEM((_LANES, dim), jnp.float32)] * _NBUF,
            [pltpu.VMEM((fgroups, 8, _LANES), jnp.float32)] * _NBUF,
            [pltpu.SemaphoreType.DMA] * _NBUF,
            [pltpu.SemaphoreType.DMA] * _NBUF,
        ],
    )
    def gather_kernel(table_hbm, idx_hbm, out_hbm, idx_blk, row_bufs,
                      t_bufs, gsems, ssems):
        wid = lax.axis_index("s") * _NC + lax.axis_index("c")
        b0 = wid * b_per_w
        bb0 = wid * blocks_per_w

        pltpu.sync_copy(idx_hbm.at[:, pl.ds(b0, b_per_w)], idx_blk)

        lanes = [lax.iota(jnp.int32, 16) + 16 * g for g in range(8)]

        def start(t, b):
            h = t // blocks_per_w
            bbl = t % blocks_per_w
            pltpu.async_copy(
                table_hbm.at[idx_blk.at[h, pl.ds(bbl * _LANES, _LANES)]],
                row_bufs[b], gsems[b])

        def drain_stores(b):
            for fb in range(fgroups):
                pltpu.make_async_copy(
                    t_bufs[b].at[fb], out_hbm.at[0, fb, 0], ssems[b]).wait()

        def do_finish(t, b, drain):
            h = t // blocks_per_w
            bb = bb0 + t % blocks_per_w
            pltpu.make_async_copy(
                table_hbm.at[idx_blk.at[h, pl.ds(0, _LANES)]],
                row_bufs[b], gsems[b]).wait()
            if drain:
                drain_stores(b)
            for f in range(dim):
                col = jnp.full((16,), f, jnp.int32)
                vals = [plsc.load_gather(row_bufs[b], [lanes[g], col])
                        for g in range(8)]
                for g in range(8):
                    t_bufs[b][f // 8, f % 8, pl.ds(16 * g, 16)] = vals[g]
            for fb in range(fgroups):
                pltpu.async_copy(
                    t_bufs[b].at[fb], out_hbm.at[h, fb, bb], ssems[b])

        start(0, 0)
        start(1, 1)
        do_finish(0, 0, False)
        start(2, 0)
        do_finish(1, 1, False)
        start(3, 1)

        def body(i, carry):
            for b in range(_NBUF):
                t = 2 + i * _NBUF + b
                do_finish(t, b, True)
                start(t + _NBUF, b)
            return carry

        lax.fori_loop(0, (n_chunks - 4) // _NBUF, body, 0)

        for b in range(_NBUF):
            do_finish(n_chunks - _NBUF + b, b, True)
        for b in range(_NBUF):
            drain_stores(b)

    return gather_kernel


def kernel(indices, table):
    nrows, hist = indices.shape
    vocab, dim = table.shape
    out5 = _build(vocab, dim, nrows, hist)(table, indices.T)
    return out5.transpose(2, 4, 0, 1, 3).reshape(nrows, hist, dim)


# speedup vs baseline: 79.5492x; 1.1631x over previous
import functools

import jax
import jax.numpy as jnp
from jax import lax
from jax.experimental import pallas as pl
from jax.experimental.pallas import tpu as pltpu
from jax.experimental.pallas import tpu_sc as plsc

_NC = 2
_NS = 16
_NW = _NC * _NS
_LANES = 128
_NBUF = 2


@functools.lru_cache(maxsize=None)
def _build(vocab, dim, nrows, hist):
    n_blocks = nrows // _LANES
    blocks_per_w = n_blocks // _NW
    b_per_w = nrows // _NW
    n_chunks = hist * blocks_per_w
    fgroups = dim // 8

    mesh = plsc.VectorSubcoreMesh(
        core_axis_name="c", subcore_axis_name="s",
        num_cores=_NC, num_subcores=_NS)

    @functools.partial(
        pl.kernel,
        out_type=jax.ShapeDtypeStruct(
            (hist, fgroups, n_blocks, 8, _LANES), jnp.float32),
        mesh=mesh,
        compiler_params=pltpu.CompilerParams(
            use_tc_tiling_on_sc=False, needs_layout_passes=False),
        scratch_types=[
            pltpu.VMEM((hist, b_per_w), jnp.int32),
            [pltpu.VMEM((_LANES, dim), jnp.float32)] * _NBUF,
            [pltpu.VMEM((fgroups, 8, _LANES), jnp.float32)] * _NBUF,
            [pltpu.SemaphoreType.DMA] * _NBUF,
            [pltpu.SemaphoreType.DMA] * _NBUF,
        ],
    )
    def gather_kernel(table_hbm, idx_hbm, out_hbm, idx_blk, row_bufs,
                      t_bufs, gsems, ssems):
        wid = lax.axis_index("s") * _NC + lax.axis_index("c")
        b0 = wid * b_per_w
        bb0 = wid * blocks_per_w

        pltpu.sync_copy(idx_hbm.at[:, pl.ds(b0, b_per_w)], idx_blk)

        lanes = [lax.iota(jnp.int32, 16) + 16 * g for g in range(8)]

        def start(t, b):
            h = t // blocks_per_w
            bbl = t % blocks_per_w
            pltpu.async_copy(
                table_hbm.at[idx_blk.at[h, pl.ds(bbl * _LANES, _LANES)]],
                row_bufs[b], gsems[b])

        def drain_stores(b):
            for fb in range(fgroups):
                pltpu.make_async_copy(
                    t_bufs[b].at[fb], out_hbm.at[0, fb, 0], ssems[b]).wait()

        def do_finish(t, b, drain):
            h = t // blocks_per_w
            bb = bb0 + t % blocks_per_w
            pltpu.make_async_copy(
                table_hbm.at[idx_blk.at[h, pl.ds(0, _LANES)]],
                row_bufs[b], gsems[b]).wait()
            if drain:
                drain_stores(b)
            for g in range(8):
                for fh in range(dim // 16):
                    vs = [plsc.load_gather(
                              row_bufs[b],
                              [lanes[g],
                               jnp.full((16,), fh * 16 + k, jnp.int32)])
                          for k in range(16)]
                    for k in range(16):
                        f = fh * 16 + k
                        t_bufs[b][f // 8, f % 8, pl.ds(16 * g, 16)] = vs[k]
            for fb in range(fgroups):
                pltpu.async_copy(
                    t_bufs[b].at[fb], out_hbm.at[h, fb, bb], ssems[b])

        start(0, 0)
        start(1, 1)
        do_finish(0, 0, False)
        start(2, 0)
        do_finish(1, 1, False)
        start(3, 1)

        def body(i, carry):
            for b in range(_NBUF):
                t = 2 + i * _NBUF + b
                do_finish(t, b, True)
                start(t + _NBUF, b)
            return carry

        lax.fori_loop(0, (n_chunks - 4) // _NBUF, body, 0)

        for b in range(_NBUF):
            do_finish(n_chunks - _NBUF + b, b, True)
        for b in range(_NBUF):
            drain_stores(b)

    return gather_kernel


def kernel(indices, table):
    nrows, hist = indices.shape
    vocab, dim = table.shape
    out5 = _build(vocab, dim, nrows, hist)(table, indices.T)
    return out5.transpose(2, 4, 0, 1, 3).reshape(nrows, hist, dim)
